# Initial kernel scaffold; baseline (speedup 1.0000x reference)
import jax
import jax.numpy as jnp
from jax.experimental import pallas as pl


def kernel(x, edge_attr, edge_aux, edge_index, Wh, bh, Wm, bm, gm, betam, Watt, Wdh, bdh, gdh, betadh, Wde, bde, gde, betade):
    raise NotImplementedError("write your pallas kernel here")


# speedup vs baseline: 3.1806x; 3.1806x over previous
import functools

import jax
import jax.numpy as jnp
from jax import lax
from jax.experimental import pallas as pl
from jax.experimental.pallas import tpu as pltpu
from jax.experimental.pallas import tpu_sc as plsc

N = 50000
E = 800000
DA = 128
DE = 64
H = 4

EROWS = E // 128
ROWPAD = 6283
NW = 32
UNITS = 782
GUPW, GREM = divmod(UNITS, NW)
SUPW, SREM = divmod(UNITS, 16)
QTR = 12512
NPAD = 4 * QTR
FCH = 32
FCHUNKS = QTR // FCH
FPW, FREM = divmod(FCHUNKS, 16)
SSEG = 56
NSEG = -(-(SUPW + 1) * 8 // SSEG)


def _silu(z):
    return z * (1.0 / (1.0 + jnp.exp(-z)))


def _down_body(x_ref, wht_ref, bh_ref, o_ref):
    z = jnp.dot(x_ref[...], wht_ref[...], preferred_element_type=jnp.float32)
    o_ref[...] = _silu(z + bh_ref[...])


def _node_down(x, WhT, bh2):
    return pl.pallas_call(
        _down_body,
        grid=(25,),
        in_specs=[
            pl.BlockSpec((2000, DA), lambda i: (i, 0)),
            pl.BlockSpec((DA, DE), lambda i: (0, 0)),
            pl.BlockSpec((1, DE), lambda i: (0, 0)),
        ],
        out_specs=pl.BlockSpec((2000, DE), lambda i: (i, 0)),
        out_shape=jax.ShapeDtypeStruct((N, DE), jnp.float32),
    )(x, WhT, bh2)


def _sc_gather(x_down, src2d, dst2d):
    mesh = plsc.VectorSubcoreMesh(core_axis_name="c", subcore_axis_name="s")

    @functools.partial(
        pl.kernel,
        mesh=mesh,
        compiler_params=pltpu.CompilerParams(use_tc_tiling_on_sc=False),
        out_type=[
            jax.ShapeDtypeStruct((E, DE), jnp.float32),
            jax.ShapeDtypeStruct((E, DE), jnp.float32),
        ],
        scratch_types=[
            pltpu.VMEM(((GUPW + 1) * 8, 128), jnp.int32),
            pltpu.VMEM(((GUPW + 1) * 8, 128), jnp.int32),
            pltpu.VMEM((128, DE), jnp.float32),
            pltpu.VMEM((128, DE), jnp.float32),
            pltpu.VMEM((128, DE), jnp.float32),
            pltpu.VMEM((128, DE), jnp.float32),
            pltpu.SemaphoreType.DMA,
            pltpu.SemaphoreType.DMA,
            pltpu.SemaphoreType.DMA,
            pltpu.SemaphoreType.DMA,
        ],
    )
    def k(xdn, srcr, dstr, xj_out, xi_out, sidx, didx,
          s0, d0, s1, d1, ss0, sd0, ss1, sd1):
        c = lax.axis_index("c")
        s = lax.axis_index("s")
        wid = c * 16 + s
        base = (wid * GUPW + jnp.minimum(wid, GREM)) * 8
        nrows = (GUPW + jnp.where(wid < GREM, 1, 0)) * 8
        nrows = jnp.maximum(jnp.minimum(base + nrows, EROWS) - base, 0)
        pltpu.sync_copy(srcr.at[pl.ds(base, (GUPW + 1) * 8), :], sidx)
        pltpu.sync_copy(dstr.at[pl.ds(base, (GUPW + 1) * 8), :], didx)

        def fire0(j):
            pltpu.async_copy(xdn.at[sidx.at[j]], s0, ss0)
            pltpu.async_copy(xdn.at[didx.at[j]], d0, sd0)

        def fire1(j):
            pltpu.async_copy(xdn.at[sidx.at[j]], s1, ss1)
            pltpu.async_copy(xdn.at[didx.at[j]], d1, sd1)

        @pl.when(nrows > 0)
        def _():
            fire0(0)

        def body(jj, carry):
            j0 = 2 * jj
            j1 = 2 * jj + 1

            @pl.when(j0 < nrows)
            def _():
                pltpu.make_async_copy(xdn.at[sidx.at[0]], s0, ss0).wait()
                pltpu.make_async_copy(xdn.at[didx.at[0]], d0, sd0).wait()

                @pl.when(j1 < nrows)
                def _():
                    fire1(j1)

                erow = (base + j0) * 128
                pltpu.sync_copy(s0, xj_out.at[pl.ds(erow, 128), :])
                pltpu.sync_copy(d0, xi_out.at[pl.ds(erow, 128), :])

            @pl.when(j1 < nrows)
            def _():
                pltpu.make_async_copy(xdn.at[sidx.at[0]], s1, ss1).wait()
                pltpu.make_async_copy(xdn.at[didx.at[0]], d1, sd1).wait()

                @pl.when(j1 + 1 < nrows)
                def _():
                    fire0(j1 + 1)

                erow = (base + j1) * 128
                pltpu.sync_copy(s1, xj_out.at[pl.ds(erow, 128), :])
                pltpu.sync_copy(d1, xi_out.at[pl.ds(erow, 128), :])

            return carry

        lax.fori_loop(0, (nrows + 1) // 2, body, 0)

    return k(x_down, src2d, dst2d)


def _edgeA_body(xi_ref, xj_ref, ea_ref, ex_ref, wmt_ref, watt_ref,
                y_ref, alpha_ref, stats_ref):
    xi = xi_ref[...]
    xj = xj_ref[...]
    ea = ea_ref[...]
    mpre = xi * xj * ea
    y = jnp.dot(mpre, wmt_ref[...], preferred_element_type=jnp.float32)
    y_ref[...] = y
    s1 = jnp.sum(y, axis=0, keepdims=True)
    s2 = jnp.sum(y * y, axis=0, keepdims=True)
    upd = jnp.concatenate([s1, s2, jnp.zeros((6, DE), jnp.float32)], axis=0)

    @pl.when(pl.program_id(0) == 0)
    def _():
        stats_ref[...] = jnp.zeros_like(stats_ref)

    stats_ref[...] += upd

    w = ea * ex_ref[...]
    vs = []
    for h in range(H):
        ai = jnp.dot(xi, watt_ref[h], preferred_element_type=jnp.float32)
        aj = jnp.dot(xj, watt_ref[h], preferred_element_type=jnp.float32)
        vs.append(jnp.mean(ai * aj * w, axis=1, keepdims=True))
    v = jnp.concatenate(vs, axis=1)
    alpha_ref[...] = v - jnp.tanh(v)


def _edge_a(xi, xj, ea, ex, WmT, WattT):
    R = 3200
    return pl.pallas_call(
        _edgeA_body,
        grid=(E // R,),
        in_specs=[
            pl.BlockSpec((R, DE), lambda i: (i, 0)),
            pl.BlockSpec((R, DE), lambda i: (i, 0)),
            pl.BlockSpec((R, DE), lambda i: (i, 0)),
            pl.BlockSpec((R, DE), lambda i: (i, 0)),
            pl.BlockSpec((DE, DE), lambda i: (0, 0)),
            pl.BlockSpec((H, DE, DE), lambda i: (0, 0, 0)),
        ],
        out_specs=[
            pl.BlockSpec((R, DE), lambda i: (i, 0)),
            pl.BlockSpec((R, H), lambda i: (i, 0)),
            pl.BlockSpec((8, DE), lambda i: (0, 0)),
        ],
        out_shape=[
            jax.ShapeDtypeStruct((E, DE), jnp.float32),
            jax.ShapeDtypeStruct((E, H), jnp.float32),
            jax.ShapeDtypeStruct((8, DE), jnp.float32),
        ],
    )(xi, xj, ea, ex, WmT, WattT)


def _edgeB_body(y_ref, alpha_ref, wdet_ref, wdht_ref, smtm_ref,
                epre_ref, q_ref, estats_ref):
    y = y_ref[...]
    m = _silu(smtm_ref[0:1, :] * y + smtm_ref[1:2, :])
    epre = jnp.dot(m, wdet_ref[...], preferred_element_type=jnp.float32)
    epre_ref[...] = epre
    s1 = jnp.sum(epre, axis=0, keepdims=True)
    s2 = jnp.sum(epre * epre, axis=0, keepdims=True)
    upd = jnp.concatenate([s1, s2, jnp.zeros((6, DE), jnp.float32)], axis=0)

    @pl.when(pl.program_id(0) == 0)
    def _():
        estats_ref[...] = jnp.zeros_like(estats_ref)

    estats_ref[...] += upd

    alpha = alpha_ref[...]
    u = jnp.concatenate([alpha[:, h:h + 1] * m for h in range(H)], axis=1)
    q_ref[...] = jnp.dot(u, wdht_ref[...], preferred_element_type=jnp.float32)


def _edge_b(y, alpha, WdeT, WdhT, smtm):
    R = 3200
    return pl.pallas_call(
        _edgeB_body,
        grid=(E // R,),
        in_specs=[
            pl.BlockSpec((R, DE), lambda i: (i, 0)),
            pl.BlockSpec((R, H), lambda i: (i, 0)),
            pl.BlockSpec((DE, DE), lambda i: (0, 0)),
            pl.BlockSpec((H * DE, DA), lambda i: (0, 0)),
            pl.BlockSpec((2, DE), lambda i: (0, 0)),
        ],
        out_specs=[
            pl.BlockSpec((R, DE), lambda i: (i, 0)),
            pl.BlockSpec((R, DA), lambda i: (i, 0)),
            pl.BlockSpec((8, DE), lambda i: (0, 0)),
        ],
        out_shape=[
            jax.ShapeDtypeStruct((E, DE), jnp.float32),
            jax.ShapeDtypeStruct((E, DA), jnp.float32),
            jax.ShapeDtypeStruct((8, DE), jnp.float32),
        ],
    )(y, alpha, WdeT, WdhT, smtm)


def _sc_scatter(q, dst2d):
    mesh = plsc.VectorSubcoreMesh(core_axis_name="c", subcore_axis_name="s")

    @functools.partial(
        pl.kernel,
        mesh=mesh,
        compiler_params=pltpu.CompilerParams(needs_layout_passes=False),
        out_type=jax.ShapeDtypeStruct((NPAD, DA), jnp.float32),
        scratch_types=[
            pltpu.VMEM_SHARED((QTR + 8, DA), jnp.float32),
            pltpu.VMEM((SSEG, 128), jnp.int32),
            pltpu.VMEM((64, DA), jnp.float32),
            pltpu.VMEM((2, 64), jnp.int32),
            pltpu.VMEM((2, 64), jnp.int32),
            pltpu.VMEM((FCH, DA), jnp.float32),
            pltpu.SemaphoreType.DMA,
        ],
    )
    def k(q_hbm, dstr, outp, acc, dstrows, qbuf, ebuf, lbuf, fbuf, gsem):
        c = lax.axis_index("c")
        s = lax.axis_index("s")
        base = (s * SUPW + jnp.minimum(s, SREM)) * 8
        nrows = (SUPW + jnp.where(s < SREM, 1, 0)) * 8
        nrows = jnp.maximum(jnp.minimum(base + nrows, EROWS) - base, 0)
        cbase = s * FPW + jnp.minimum(s, FREM)
        nch = FPW + jnp.where(s < FREM, 1, 0)

        z16 = jnp.zeros((16,), jnp.float32)
        zi16 = jnp.zeros((16,), jnp.int32)
        iota = lax.iota(jnp.int32, 16)

        def zrow(i, carry):
            for t in range(8):
                fbuf[i, pl.ds(t * 16, 16)] = z16
            return carry

        def flush(slot):
            pltpu.async_copy(q_hbm.at[ebuf.at[slot]], qbuf, gsem).wait()
            pltpu.sync_copy(qbuf, acc.at[lbuf.at[slot]], add=True)

        for r in range(2):
            qtr = 2 * c + r
            lo = qtr * QTR
            lax.fori_loop(0, FCH, zrow, 0)

            def zch(i, carry):
                pltpu.sync_copy(
                    fbuf, acc.at[pl.ds((cbase + i) * FCH, FCH), :])
                return carry

            lax.fori_loop(0, nch, zch, 0)
            for sl in range(2):
                for t in range(4):
                    ebuf[sl, pl.ds(t * 16, 16)] = zi16
                    lbuf[sl, pl.ds(t * 16, 16)] = zi16 + QTR

            plsc.subcore_barrier()

            def seg(g, carry):
                pltpu.sync_copy(
                    dstr.at[pl.ds(base + g * SSEG, SSEG), :], dstrows)
                nj = jnp.maximum(jnp.minimum(nrows - g * SSEG, SSEG), 0)

                def body(j, cn):
                    cnt, nfl = cn
                    ebase = (base + g * SSEG + j) * 128
                    for t in range(8):
                        d = dstrows[j, pl.ds(t * 16, 16)]
                        msk = (d >= lo) & (d < lo + QTR)
                        inc = jnp.where(msk, 1, 0).astype(jnp.int32)
                        pos = cnt + jnp.cumsum(inc) - 1
                        pr = lax.rem(pos, 128)
                        row = lax.shift_right_logical(pr, 6)
                        col = lax.rem(pr, 64)
                        eid = ebase + t * 16 + iota
                        plsc.store_scatter(ebuf, [row, col], eid, mask=msk)
                        plsc.store_scatter(
                            lbuf, [row, col], d - lo, mask=msk)
                        cnt = cnt + jnp.sum(inc)
                        do_flush = cnt - nfl >= 64
                        slot = lax.rem(lax.shift_right_logical(nfl, 6), 2)

                        @pl.when(do_flush)
                        def _():
                            flush(slot)

                        nfl = nfl + jnp.where(do_flush, 64, 0)
                    return (cnt, nfl)

                return lax.fori_loop(0, nj, body, carry)

            cnt, nfl = lax.fori_loop(
                0, NSEG, seg, (jnp.int32(0), jnp.int32(0)))
            rem = cnt - nfl

            @pl.when(rem > 0)
            def _():
                slot = lax.rem(lax.shift_right_logical(nfl, 6), 2)
                for t in range(4):
                    lane = t * 16 + iota
                    cur = lbuf[slot, pl.ds(t * 16, 16)]
                    lbuf[slot, pl.ds(t * 16, 16)] = jnp.where(
                        lane < rem, cur, QTR)
                flush(slot)

            plsc.subcore_barrier()

            def fch(i, carry):
                off = (cbase + i) * FCH
                pltpu.sync_copy(acc.at[pl.ds(off, FCH), :], fbuf)
                pltpu.sync_copy(fbuf, outp.at[pl.ds(lo + off, FCH), :])
                return carry

            lax.fori_loop(0, nch, fch, 0)
            plsc.subcore_barrier()

    return k(q, dst2d)


def _ostats_body(op_ref, stats_ref):
    op = op_ref[...]
    s1 = jnp.sum(op, axis=0, keepdims=True)
    s2 = jnp.sum(op * op, axis=0, keepdims=True)
    upd = jnp.concatenate([s1, s2, jnp.zeros((6, DA), jnp.float32)], axis=0)

    @pl.when(pl.program_id(0) == 0)
    def _():
        stats_ref[...] = jnp.zeros_like(stats_ref)

    stats_ref[...] += upd


def _out_stats(outp):
    return pl.pallas_call(
        _ostats_body,
        grid=(16,),
        in_specs=[pl.BlockSpec((NPAD // 16, DA), lambda i: (i, 0))],
        out_specs=pl.BlockSpec((8, DA), lambda i: (0, 0)),
        out_shape=jax.ShapeDtypeStruct((8, DA), jnp.float32),
    )(outp)


def _final_body(op_ref, x_ref, soto_ref, o_ref):
    o = _silu(soto_ref[0:1, :] * op_ref[...] + soto_ref[1:2, :])
    o_ref[...] = x_ref[...] + o


def _final_x(outp, x, soto):
    return pl.pallas_call(
        _final_body,
        grid=(25,),
        in_specs=[
            pl.BlockSpec((2000, DA), lambda i: (i, 0)),
            pl.BlockSpec((2000, DA), lambda i: (i, 0)),
            pl.BlockSpec((2, DA), lambda i: (0, 0)),
        ],
        out_specs=pl.BlockSpec((2000, DA), lambda i: (i, 0)),
        out_shape=jax.ShapeDtypeStruct((N, DA), jnp.float32),
    )(outp, x, soto)


def _e_body(ep_ref, sete_ref, e_ref):
    e_ref[...] = _silu(sete_ref[0:1, :] * ep_ref[...] + sete_ref[1:2, :])


def _final_e(epre, sete):
    return pl.pallas_call(
        _e_body,
        grid=(100,),
        in_specs=[
            pl.BlockSpec((8000, DE), lambda i: (i, 0)),
            pl.BlockSpec((2, DE), lambda i: (0, 0)),
        ],
        out_specs=pl.BlockSpec((8000, DE), lambda i: (i, 0)),
        out_shape=jax.ShapeDtypeStruct((E, DE), jnp.float32),
    )(epre, sete)


def _bn_coeffs(s1, s2, count, g, beta):
    mu = s1 / count
    var = s2 / count - mu * mu
    s = g / jnp.sqrt(var + 1e-5)
    return s, beta - mu * s


def kernel(x, edge_attr, edge_aux, edge_index, Wh, bh, Wm, bm, gm, betam,
           Watt, Wdh, bdh, gdh, betadh, Wde, bde, gde, betade):
    ei = edge_index.astype(jnp.int32)
    src2d = jnp.pad(ei[0].reshape(EROWS, 128), ((0, NW + 1), (0, 0)))
    dst2d = jnp.pad(ei[1].reshape(EROWS, 128), ((0, NW + 1), (0, 0)))

    x_down = _node_down(x, Wh.T, bh.reshape(1, DE))
    xj, xi = _sc_gather(x_down, src2d, dst2d)

    y, alpha, mstats = _edge_a(xi, xj, edge_attr, edge_aux, Wm.T,
                               jnp.transpose(Watt, (0, 2, 1)))
    sm, tm = _bn_coeffs(mstats[0], mstats[1], E, gm, betam)
    smtm = jnp.stack([sm, tm])

    epre, q, estats = _edge_b(y, alpha, Wde.T, Wdh.T, smtm)
    se, te = _bn_coeffs(estats[0], estats[1], E, gde, betade)
    e = _final_e(epre, jnp.stack([se, te]))

    outp = _sc_scatter(q, dst2d)
    ostats = _out_stats(outp)
    so, to = _bn_coeffs(ostats[0], ostats[1], N, gdh, betadh)
    x_out = _final_x(outp, x, jnp.stack([so, to]))
    return (x_out, e)
